# Initial kernel scaffold; baseline (speedup 1.0000x reference)
import jax
import jax.numpy as jnp
from jax.experimental import pallas as pl


def kernel(x, table):
    raise NotImplementedError("write your pallas kernel here")


# speedup vs baseline: 1.8431x; 1.8431x over previous
import functools

import jax
import jax.numpy as jnp
from jax import lax
from jax.experimental import pallas as pl
from jax.experimental.pallas import tpu as pltpu
from jax.experimental.pallas import tpu_sc as plsc

D_EMBED = 64
NC = 2
NS = 16
NW = NC * NS
CH = 128
K = 8


@functools.partial(jax.jit, static_argnames=("total_rows",))
def _emb_lookup(xf, table, total_rows):
    b_per_w = total_rows // NW
    n_groups = b_per_w // (CH * K)
    mesh = plsc.VectorSubcoreMesh(core_axis_name="c", subcore_axis_name="s")

    @functools.partial(
        pl.kernel,
        mesh=mesh,
        out_type=jax.ShapeDtypeStruct((total_rows, D_EMBED), jnp.float32),
        scratch_types=[
            pltpu.VMEM((CH * K,), jnp.int32),
            pltpu.VMEM((CH * K, D_EMBED), jnp.float32),
            pltpu.SemaphoreType.DMA,
        ],
        compiler_params=pltpu.CompilerParams(use_tc_tiling_on_sc=False),
    )
    def emb(x_hbm, table_hbm, out_hbm, idx_v, rows_v, sem):
        wid = lax.axis_index("s") * NC + lax.axis_index("c")
        base = wid * b_per_w

        def body(g, carry):
            gbase = base + g * (CH * K)
            pltpu.sync_copy(x_hbm.at[pl.ds(gbase, CH * K)], idx_v)
            copies = []
            for j in range(K):
                copies.append(pltpu.async_copy(
                    table_hbm.at[idx_v.at[pl.ds(j * CH, CH)]],
                    rows_v.at[pl.ds(j * CH, CH)],
                    sem,
                ))
            for c in copies:
                c.wait()
            pltpu.sync_copy(rows_v, out_hbm.at[pl.ds(gbase, CH * K)])
            return carry

        lax.fori_loop(0, n_groups, body, 0)

    return emb(xf, table)


def kernel(x, table):
    total_rows = x.shape[0] * x.shape[1]
    xf = x.reshape(-1).astype(jnp.int32)
    out = _emb_lookup(xf, table, total_rows)
    return out.reshape(x.shape[0], x.shape[1], D_EMBED)
